# Initial kernel scaffold; baseline (speedup 1.0000x reference)
import jax
import jax.numpy as jnp
from jax.experimental import pallas as pl


def kernel(x, l0_wih, l0_whhf, l0_whhb, l0_bih, l0_bhn, l1_wih, l1_whhf, l1_whhb, l1_bih, l1_bhn, lin_w, lin_b):
    raise NotImplementedError("write your pallas kernel here")


# speedup vs baseline: 2.2819x; 2.2819x over previous
import functools

import jax
import jax.numpy as jnp
from jax.experimental import pallas as pl
from jax.experimental.pallas import tpu as pltpu

_VMEM_LIMIT = 64 * 1024 * 1024


def _fused_gru_kernel(x_ref, w0_ref, whhf0_ref, whhb0_ref, bih0_ref, bhn0_ref,
                      w1_ref, whhf1_ref, whhb1_ref, bih1_ref, bhn1_ref,
                      wlin_ref, blin_ref, out_ref, y0_ref, *, hidden, inv_denom):
    T, BT, _ = x_ref.shape
    H = hidden
    G = 3 * H

    def make_step(in_ref, wih_ref, whhf_ref, whhb_ref, bih_ref, bhn_ref):
        bih_f = bih_ref[:, :G]
        bih_b = bih_ref[:, G:]
        bhn_f = bhn_ref[:, :H]
        bhn_b = bhn_ref[:, H:]
        wih_f = wih_ref[:, :G]
        wih_b = wih_ref[:, G:]

        def gates(gi, gh, h, bhn):
            r = jax.nn.sigmoid(gi[:, :H] + gh[:, :H])
            z = jax.nn.sigmoid(gi[:, H:2 * H] + gh[:, H:2 * H])
            n = jnp.tanh(gi[:, 2 * H:] + r * (gh[:, 2 * H:] + bhn))
            return (1.0 - z) * n + z * h

        def step(i, carry):
            hf, hb = carry
            t = i
            rt = T - 1 - i
            gif = jnp.dot(in_ref[t], wih_f,
                          preferred_element_type=jnp.float32) + bih_f
            gib = jnp.dot(in_ref[rt], wih_b,
                          preferred_element_type=jnp.float32) + bih_b
            ghf = jnp.dot(hf.astype(jnp.bfloat16), whhf_ref[...],
                          preferred_element_type=jnp.float32)
            ghb = jnp.dot(hb.astype(jnp.bfloat16), whhb_ref[...],
                          preferred_element_type=jnp.float32)
            hf_new = gates(gif, ghf, hf, bhn_f)
            hb_new = gates(gib, ghb, hb, bhn_b)
            return t, rt, hf_new, hb_new

        return step

    h0 = jnp.zeros((BT, H), jnp.float32)

    step0 = make_step(x_ref, w0_ref, whhf0_ref, whhb0_ref, bih0_ref, bhn0_ref)

    def body0(i, carry):
        t, rt, hf, hb = step0(i, carry)
        y0_ref[t, :, :H] = hf.astype(jnp.bfloat16)
        y0_ref[rt, :, H:] = hb.astype(jnp.bfloat16)
        return hf, hb

    jax.lax.fori_loop(0, T, body0, (h0, h0), unroll=2)

    step1 = make_step(y0_ref, w1_ref, whhf1_ref, whhb1_ref, bih1_ref, bhn1_ref)

    def body1(i, carry):
        hf, hb, acc = carry
        _, _, hf_new, hb_new = step1(i, (hf, hb))
        return hf_new, hb_new, acc + hf_new + hb_new

    _, _, acc = jax.lax.fori_loop(0, T, body1, (h0, h0, h0), unroll=2)

    s = acc * inv_denom
    logits = jnp.dot(s.astype(jnp.bfloat16), wlin_ref[...],
                     preferred_element_type=jnp.float32) + blin_ref[...]
    m = jnp.max(logits, axis=1, keepdims=True)
    lse = jnp.log(jnp.sum(jnp.exp(logits - m), axis=1, keepdims=True)) + m
    out_ref[...] = logits - lse


def kernel(x, l0_wih, l0_whhf, l0_whhb, l0_bih, l0_bhn,
           l1_wih, l1_whhf, l1_whhb, l1_bih, l1_bhn, lin_w, lin_b):
    T, B, E = x.shape
    H = l0_whhf.shape[0]
    Cp = lin_w.shape[1]
    C = 1000

    bt = B // 2
    x_bf = x.astype(jnp.bfloat16)

    kernel_fn = functools.partial(_fused_gru_kernel, hidden=H,
                                  inv_denom=1.0 / (2.0 * T))
    out = pl.pallas_call(
        kernel_fn,
        out_shape=jax.ShapeDtypeStruct((B, Cp), jnp.float32),
        grid=(B // bt,),
        in_specs=[
            pl.BlockSpec((T, bt, E), lambda b: (0, b, 0)),
            pl.BlockSpec(l0_wih.shape, lambda b: (0, 0)),
            pl.BlockSpec(l0_whhf.shape, lambda b: (0, 0)),
            pl.BlockSpec(l0_whhb.shape, lambda b: (0, 0)),
            pl.BlockSpec(l0_bih.shape, lambda b: (0, 0)),
            pl.BlockSpec(l0_bhn.shape, lambda b: (0, 0)),
            pl.BlockSpec(l1_wih.shape, lambda b: (0, 0)),
            pl.BlockSpec(l1_whhf.shape, lambda b: (0, 0)),
            pl.BlockSpec(l1_whhb.shape, lambda b: (0, 0)),
            pl.BlockSpec(l1_bih.shape, lambda b: (0, 0)),
            pl.BlockSpec(l1_bhn.shape, lambda b: (0, 0)),
            pl.BlockSpec(lin_w.shape, lambda b: (0, 0)),
            pl.BlockSpec(lin_b.shape, lambda b: (0, 0)),
        ],
        out_specs=pl.BlockSpec((bt, Cp), lambda b: (b, 0)),
        scratch_shapes=[pltpu.VMEM((T, bt, 2 * H), jnp.bfloat16)],
        compiler_params=pltpu.CompilerParams(
            dimension_semantics=("parallel",),
            vmem_limit_bytes=_VMEM_LIMIT),
    )(x_bf, l0_wih, l0_whhf, l0_whhb, l0_bih, l0_bhn,
      l1_wih, l1_whhf, l1_whhb, l1_bih, l1_bhn, lin_w, lin_b)
    return out[:, :C]


# speedup vs baseline: 2.5018x; 1.0964x over previous
import functools

import jax
import jax.numpy as jnp
from jax.experimental import pallas as pl
from jax.experimental.pallas import tpu as pltpu

_VMEM_LIMIT = 64 * 1024 * 1024


def _fused_gru_kernel(x_ref, w0_ref, whhf0_ref, whhb0_ref, bih0_ref, bhn0_ref,
                      w1_ref, whhf1_ref, whhb1_ref, bih1_ref, bhn1_ref,
                      wlin_ref, blin_ref, out_ref, y0_ref, *, hidden, inv_denom):
    T, BT, _ = x_ref.shape
    H = hidden
    G = 3 * H

    def make_step(in_ref, wih_ref, whhf_ref, whhb_ref, bih_ref, bhn_ref):
        bih_f = bih_ref[:, :G]
        bih_b = bih_ref[:, G:]
        bhn_f = bhn_ref[:, :H]
        bhn_b = bhn_ref[:, H:]
        wih_f = wih_ref[:, :G]
        wih_b = wih_ref[:, G:]

        def gates(gi, gh, h, bhn):
            r = jax.nn.sigmoid(gi[:, :H] + gh[:, :H])
            z = jax.nn.sigmoid(gi[:, H:2 * H] + gh[:, H:2 * H])
            n = jnp.tanh(gi[:, 2 * H:] + r * (gh[:, 2 * H:] + bhn))
            return (1.0 - z) * n + z * h

        def step(i, carry):
            hf, hb = carry
            t = i
            rt = T - 1 - i
            gif = jnp.dot(in_ref[t], wih_f,
                          preferred_element_type=jnp.float32) + bih_f
            gib = jnp.dot(in_ref[rt], wih_b,
                          preferred_element_type=jnp.float32) + bih_b
            ghf = jnp.dot(hf.astype(jnp.bfloat16), whhf_ref[...],
                          preferred_element_type=jnp.float32)
            ghb = jnp.dot(hb.astype(jnp.bfloat16), whhb_ref[...],
                          preferred_element_type=jnp.float32)
            hf_new = gates(gif, ghf, hf, bhn_f)
            hb_new = gates(gib, ghb, hb, bhn_b)
            return t, rt, hf_new, hb_new

        return step

    h0 = jnp.zeros((BT, H), jnp.float32)

    step0 = make_step(x_ref, w0_ref, whhf0_ref, whhb0_ref, bih0_ref, bhn0_ref)

    def body0(i, carry):
        t, rt, hf, hb = step0(i, carry)
        y0_ref[t, :, :H] = hf.astype(jnp.bfloat16)
        y0_ref[rt, :, H:] = hb.astype(jnp.bfloat16)
        return hf, hb

    jax.lax.fori_loop(0, T, body0, (h0, h0), unroll=4)

    step1 = make_step(y0_ref, w1_ref, whhf1_ref, whhb1_ref, bih1_ref, bhn1_ref)

    def body1(i, carry):
        hf, hb, acc = carry
        _, _, hf_new, hb_new = step1(i, (hf, hb))
        return hf_new, hb_new, acc + hf_new + hb_new

    _, _, acc = jax.lax.fori_loop(0, T, body1, (h0, h0, h0), unroll=4)

    s = acc * inv_denom
    logits = jnp.dot(s.astype(jnp.bfloat16), wlin_ref[...],
                     preferred_element_type=jnp.float32) + blin_ref[...]
    m = jnp.max(logits, axis=1, keepdims=True)
    lse = jnp.log(jnp.sum(jnp.exp(logits - m), axis=1, keepdims=True)) + m
    out_ref[...] = logits - lse


def kernel(x, l0_wih, l0_whhf, l0_whhb, l0_bih, l0_bhn,
           l1_wih, l1_whhf, l1_whhb, l1_bih, l1_bhn, lin_w, lin_b):
    T, B, E = x.shape
    H = l0_whhf.shape[0]
    Cp = lin_w.shape[1]
    C = 1000

    bt = B // 2
    x_bf = x.astype(jnp.bfloat16)

    kernel_fn = functools.partial(_fused_gru_kernel, hidden=H,
                                  inv_denom=1.0 / (2.0 * T))
    out = pl.pallas_call(
        kernel_fn,
        out_shape=jax.ShapeDtypeStruct((B, Cp), jnp.float32),
        grid=(B // bt,),
        in_specs=[
            pl.BlockSpec((T, bt, E), lambda b: (0, b, 0)),
            pl.BlockSpec(l0_wih.shape, lambda b: (0, 0)),
            pl.BlockSpec(l0_whhf.shape, lambda b: (0, 0)),
            pl.BlockSpec(l0_whhb.shape, lambda b: (0, 0)),
            pl.BlockSpec(l0_bih.shape, lambda b: (0, 0)),
            pl.BlockSpec(l0_bhn.shape, lambda b: (0, 0)),
            pl.BlockSpec(l1_wih.shape, lambda b: (0, 0)),
            pl.BlockSpec(l1_whhf.shape, lambda b: (0, 0)),
            pl.BlockSpec(l1_whhb.shape, lambda b: (0, 0)),
            pl.BlockSpec(l1_bih.shape, lambda b: (0, 0)),
            pl.BlockSpec(l1_bhn.shape, lambda b: (0, 0)),
            pl.BlockSpec(lin_w.shape, lambda b: (0, 0)),
            pl.BlockSpec(lin_b.shape, lambda b: (0, 0)),
        ],
        out_specs=pl.BlockSpec((bt, Cp), lambda b: (b, 0)),
        scratch_shapes=[pltpu.VMEM((T, bt, 2 * H), jnp.bfloat16)],
        compiler_params=pltpu.CompilerParams(
            dimension_semantics=("parallel",),
            vmem_limit_bytes=_VMEM_LIMIT),
    )(x_bf, l0_wih, l0_whhf, l0_whhb, l0_bih, l0_bhn,
      l1_wih, l1_whhf, l1_whhb, l1_bih, l1_bhn, lin_w, lin_b)
    return out[:, :C]
